# baseline (speedup 1.0000x reference)
import functools

import numpy as np

import jax
import jax.numpy as jnp
from jax import lax
from jax.experimental import pallas as pl
from jax.experimental.pallas import tpu as pltpu
from jax.experimental.pallas import tpu_sc as plsc

B, S, FI, K, OF = 8, 1024, 64, 32, 128
EPS = 1e-5
_NW = 32


_RT = 128


def _exchange(k, v, j, kk, iota):
    n = k.shape[0]
    pk = jnp.where((iota & j) == 0, pltpu.roll(k, n - j, 0), pltpu.roll(k, j, 0))
    pv = jnp.where((iota & j) == 0, pltpu.roll(v, n - j, 0), pltpu.roll(v, j, 0))
    lt = (k < pk) | ((k == pk) & (v < pv))
    wantmin = ((iota & j) == 0) == ((iota & kk) == 0)
    return jnp.where(lt == wantmin, k, pk), jnp.where(lt == wantmin, v, pv)


def _merge_pairs(k, v):
    r = k.shape[1]
    g = k.shape[0] // 64
    pk = pltpu.roll(k, k.shape[0] - 32, 0)
    pv = pltpu.roll(v, v.shape[0] - 32, 0)
    lt = (k < pk) | ((k == pk) & (v < pv))
    mk = jnp.where(lt, k, pk).reshape(g, 2, 32, r)
    mv = jnp.where(lt, v, pv).reshape(g, 2, 32, r)
    return mk[:, 0].reshape(-1, r), mv[:, 0].reshape(-1, r)


def _knn_body(xa_ref, xt_ref, out_ref):
    b = pl.program_id(0)
    xa = xa_ref[0]
    xt = xt_ref[0]
    sqa = jnp.sum(xa * xa, axis=1, keepdims=True)
    sqt = lax.dot_general(jnp.ones((1, FI), jnp.float32), xt * xt,
                          (((1,), (1,)), ((), ())),
                          preferred_element_type=jnp.float32,
                          precision=lax.Precision.HIGHEST)
    dot = lax.dot_general(xa, xt, (((1,), (1,)), ((), ())),
                          preferred_element_type=jnp.float32)
    key = sqa + sqt - 2.0 * dot
    val = lax.broadcasted_iota(jnp.int32, (S, _RT), 0)

    iota = lax.broadcasted_iota(jnp.int32, (S, 1), 0)
    for kk in (2, 4, 8, 16, 32):
        j = kk // 2
        while j >= 1:
            key, val = _exchange(key, val, j, kk, iota)
            j //= 2
    while key.shape[0] > K:
        key, val = _merge_pairs(key, val)
        iota = lax.broadcasted_iota(jnp.int32, (key.shape[0], 1), 0)
        for j in (16, 8, 4, 2, 1):
            key, val = _exchange(key, val, j, 32, iota)
    out_ref[0] = val + b * S


def _knn(x):
    return pl.pallas_call(
        _knn_body,
        grid=(B, S // _RT),
        in_specs=[
            pl.BlockSpec((1, S, FI), lambda b, i: (b, 0, 0)),
            pl.BlockSpec((1, _RT, FI), lambda b, i: (b, i, 0)),
        ],
        out_specs=pl.BlockSpec((1, K, _RT), lambda b, i: (b, 0, i)),
        out_shape=jax.ShapeDtypeStruct((B, K, S), jnp.int32),
    )(x, x)


_CHUNK = 128
_N = B * S * K


def _gather_body(xf_hbm, gidx_hbm, out_hbm, idx_v, rows_v, sem):
    wid = lax.axis_index("s") * 2 + lax.axis_index("c")
    n_w = _N // _NW
    base = wid * n_w

    def body(i, carry):
        off = base + i * _CHUNK
        pltpu.sync_copy(gidx_hbm.at[pl.ds(off, _CHUNK)], idx_v)
        pltpu.async_copy(xf_hbm.at[idx_v], rows_v, sem).wait()
        pltpu.sync_copy(rows_v, out_hbm.at[pl.ds(off, _CHUNK)])
        return carry

    lax.fori_loop(0, n_w // _CHUNK, body, 0)


@functools.cache
def _make_gather():
    return pl.kernel(
        _gather_body,
        out_type=jax.ShapeDtypeStruct((_N, FI), jnp.float32),
        mesh=plsc.VectorSubcoreMesh(core_axis_name="c", subcore_axis_name="s"),
        scratch_types=[
            pltpu.VMEM((_CHUNK,), jnp.int32),
            pltpu.VMEM((_CHUNK, FI), jnp.float32),
            pltpu.SemaphoreType.DMA,
        ],
        compiler_params=pltpu.CompilerParams(use_tc_tiling_on_sc=False),
    )


_SB = 16


def _mlp_body(gx_ref, xc_ref, w1a_ref, w1b_ref, w2_ref, b1_ref, b2_ref,
              gam_ref, bet_ref, out_ref, h1_ref):
    st = xc_ref[0]
    a1 = lax.dot_general(st, w1a_ref[...], (((1,), (1,)), ((), ())),
                         preferred_element_type=jnp.float32)
    nkd = (st[None] - gx_ref[...]).reshape(B * _SB * K, FI)
    a2 = lax.dot_general(nkd, w1b_ref[...], (((1,), (1,)), ((), ())),
                         preferred_element_type=jnp.float32)
    a2 = a2.reshape(B, _SB * K, OF)
    h1_ref[...] = (a1 + b1_ref[...])[None] + a2

    gam = gam_ref[...]
    bet = bet_ref[...]
    nelem = jnp.float32(B * K * OF)

    def _stats(slab):
        ssum = jnp.sum(jnp.sum(jnp.sum(slab, axis=2, keepdims=True), axis=1,
                               keepdims=True), axis=0, keepdims=True)
        ssq = jnp.sum(jnp.sum(jnp.sum(slab * slab, axis=2, keepdims=True),
                              axis=1, keepdims=True), axis=0, keepdims=True)
        m = ssum / nelem
        return m, ssq / nelem - m * m

    for sl in range(_SB):
        slab = h1_ref[:, sl * K:(sl + 1) * K, :]
        m, v = _stats(slab)
        gam_s = lax.slice(gam, (0, 0, sl), (1, 1, sl + 1))
        bet_s = lax.slice(bet, (0, 0, sl), (1, 1, sl + 1))
        a = gam_s * lax.rsqrt(v + EPS)
        h1_ref[:, sl * K:(sl + 1) * K, :] = jnp.maximum(
            (slab - m) * a + bet_s, 0.0)

    h2 = lax.dot_general(h1_ref[...].reshape(B * _SB * K, OF), w2_ref[...],
                         (((1,), (1,)), ((), ())),
                         preferred_element_type=jnp.float32) + b2_ref[...]
    h1_ref[...] = h2.reshape(B, _SB * K, OF)

    for sl in range(_SB):
        slab = h1_ref[:, sl * K:(sl + 1) * K, :]
        m, v = _stats(slab)
        gam_s = lax.slice(gam, (0, 0, sl), (1, 1, sl + 1))
        bet_s = lax.slice(bet, (0, 0, sl), (1, 1, sl + 1))
        a = gam_s * lax.rsqrt(v + EPS)
        hb2 = jnp.maximum((slab - m) * a + bet_s, 0.0)
        out_ref[:, sl, :] = jnp.max(hb2, axis=1)


def _mlp(gx, xc, w1a, w1b, w2, b1, b2, gam, bet):
    nblk = S // _SB
    return pl.pallas_call(
        _mlp_body,
        grid=(nblk,),
        in_specs=[
            pl.BlockSpec((B, _SB * K, FI), lambda i: (0, i, 0)),
            pl.BlockSpec((1, _SB * K, FI), lambda i: (i % (B * S // (_SB * K)), 0, 0)),
            pl.BlockSpec((OF, FI), lambda i: (0, 0)),
            pl.BlockSpec((OF, FI), lambda i: (0, 0)),
            pl.BlockSpec((OF, OF), lambda i: (0, 0)),
            pl.BlockSpec((1, OF), lambda i: (0, 0)),
            pl.BlockSpec((1, OF), lambda i: (0, 0)),
            pl.BlockSpec((1, 1, _SB), lambda i: (i, 0, 0)),
            pl.BlockSpec((1, 1, _SB), lambda i: (i, 0, 0)),
        ],
        out_specs=pl.BlockSpec((B, _SB, OF), lambda i: (0, i, 0)),
        out_shape=jax.ShapeDtypeStruct((B, S, OF), jnp.float32),
        scratch_shapes=[pltpu.VMEM((B, _SB * K, OF), jnp.float32)],
    )(gx, xc, w1a, w1b, w2, b1, b2, gam, bet)


def kernel(x, W1, b1, W2, b2, gamma, beta):
    xf = x.reshape(B * S, FI)
    gidx = _knn(x).transpose(0, 2, 1)
    gx = _make_gather()(xf, gidx.reshape(-1))

    w1a, w1b = W1[:, :FI], W1[:, FI:]
    out = _mlp(
        gx.reshape(B, S * K, FI),
        xf.reshape(B * S // (_SB * K), _SB * K, FI),
        w1a, w1b, W2,
        b1.reshape(1, OF), b2.reshape(1, OF),
        gamma.reshape(S // _SB, 1, _SB), beta.reshape(S // _SB, 1, _SB),
    )
    return out


# speedup vs baseline: 1.3018x; 1.3018x over previous
import functools

import numpy as np

import jax
import jax.numpy as jnp
from jax import lax
from jax.experimental import pallas as pl
from jax.experimental.pallas import tpu as pltpu
from jax.experimental.pallas import tpu_sc as plsc

B, S, FI, K, OF = 8, 1024, 64, 32, 128
EPS = 1e-5
_NW = 32


_RT = 128


def _exchange(k, v, j, kk, iota):
    n = k.shape[0]
    is_a = (iota & j) == 0
    pk = jnp.where(is_a, pltpu.roll(k, n - j, 0), pltpu.roll(k, j, 0))
    pv = jnp.where(is_a, pltpu.roll(v, n - j, 0), pltpu.roll(v, j, 0))
    lt = k < pk
    wantmin = is_a == ((iota & kk) == 0)
    keep = lt == wantmin
    return jnp.where(keep, k, pk), jnp.where(keep, v, pv)


def _merge_pairs(k, v):
    r = k.shape[1]
    g = k.shape[0] // 64
    pk = pltpu.roll(k, k.shape[0] - 32, 0)
    pv = pltpu.roll(v, v.shape[0] - 32, 0)
    lt = (k < pk) | ((k == pk) & (v < pv))
    mk = jnp.where(lt, k, pk).reshape(g, 2, 32, r)
    mv = jnp.where(lt, v, pv).reshape(g, 2, 32, r)
    return mk[:, 0].reshape(-1, r), mv[:, 0].reshape(-1, r)


def _knn_body(xa_ref, xt_ref, out_ref):
    b = pl.program_id(0)
    xa = xa_ref[0]
    xt = xt_ref[0]
    sqa = jnp.sum(xa * xa, axis=1, keepdims=True)
    sqt = lax.dot_general(jnp.ones((1, FI), jnp.float32), xt * xt,
                          (((1,), (1,)), ((), ())),
                          preferred_element_type=jnp.float32,
                          precision=lax.Precision.HIGHEST)
    dot = lax.dot_general(xa, xt, (((1,), (1,)), ((), ())),
                          preferred_element_type=jnp.float32)
    key = sqa + sqt - 2.0 * dot
    val = lax.broadcasted_iota(jnp.int32, (S, _RT), 0)

    iota = lax.broadcasted_iota(jnp.int32, (S, 1), 0)
    for kk in (2, 4, 8, 16, 32):
        j = kk // 2
        while j >= 1:
            key, val = _exchange(key, val, j, kk, iota)
            j //= 2
    while key.shape[0] > K:
        key, val = _merge_pairs(key, val)
        iota = lax.broadcasted_iota(jnp.int32, (key.shape[0], 1), 0)
        for j in (16, 8, 4, 2, 1):
            key, val = _exchange(key, val, j, 32, iota)
    out_ref[0] = val + b * S


def _knn(x):
    return pl.pallas_call(
        _knn_body,
        grid=(B, S // _RT),
        in_specs=[
            pl.BlockSpec((1, S, FI), lambda b, i: (b, 0, 0)),
            pl.BlockSpec((1, _RT, FI), lambda b, i: (b, i, 0)),
        ],
        out_specs=pl.BlockSpec((1, K, _RT), lambda b, i: (b, 0, i)),
        out_shape=jax.ShapeDtypeStruct((B, K, S), jnp.int32),
    )(x, x)


_CHUNK = 128
_N = B * S * K


def _gather_body(xf_hbm, gidx_hbm, out_hbm, idx_v, rows_v, sem):
    wid = lax.axis_index("s") * 2 + lax.axis_index("c")
    n_w = _N // _NW
    base = wid * n_w

    def body(i, carry):
        off = base + i * _CHUNK
        pltpu.sync_copy(gidx_hbm.at[pl.ds(off, _CHUNK)], idx_v)
        pltpu.async_copy(xf_hbm.at[idx_v], rows_v, sem).wait()
        pltpu.sync_copy(rows_v, out_hbm.at[pl.ds(off, _CHUNK)])
        return carry

    lax.fori_loop(0, n_w // _CHUNK, body, 0)


@functools.cache
def _make_gather():
    return pl.kernel(
        _gather_body,
        out_type=jax.ShapeDtypeStruct((_N, FI), jnp.float32),
        mesh=plsc.VectorSubcoreMesh(core_axis_name="c", subcore_axis_name="s"),
        scratch_types=[
            pltpu.VMEM((_CHUNK,), jnp.int32),
            pltpu.VMEM((_CHUNK, FI), jnp.float32),
            pltpu.SemaphoreType.DMA,
        ],
        compiler_params=pltpu.CompilerParams(use_tc_tiling_on_sc=False),
    )


_SB = 8


def _mlp_body(gx_ref, xc_ref, w1a_ref, w1b_ref, w2_ref, b1_ref, b2_ref,
              gam_ref, bet_ref, out_ref, h1_ref):
    st = xc_ref[0]
    a1 = lax.dot_general(st, w1a_ref[...], (((1,), (1,)), ((), ())),
                         preferred_element_type=jnp.float32)
    nkd = (st[None] - gx_ref[...]).reshape(B * _SB * K, FI)
    a2 = lax.dot_general(nkd, w1b_ref[...], (((1,), (1,)), ((), ())),
                         preferred_element_type=jnp.float32)
    a2 = a2.reshape(B, _SB * K, OF)
    h1_ref[...] = (a1 + b1_ref[...])[None] + a2

    gam = gam_ref[...]
    bet = bet_ref[...]
    nelem = jnp.float32(B * K * OF)

    def _stats(slab):
        ssum = jnp.sum(jnp.sum(jnp.sum(slab, axis=2, keepdims=True), axis=1,
                               keepdims=True), axis=0, keepdims=True)
        ssq = jnp.sum(jnp.sum(jnp.sum(slab * slab, axis=2, keepdims=True),
                              axis=1, keepdims=True), axis=0, keepdims=True)
        m = ssum / nelem
        return m, ssq / nelem - m * m

    for sl in range(_SB):
        slab = h1_ref[:, sl * K:(sl + 1) * K, :]
        m, v = _stats(slab)
        gam_s = lax.slice(gam, (0, 0, sl), (1, 1, sl + 1))
        bet_s = lax.slice(bet, (0, 0, sl), (1, 1, sl + 1))
        a = gam_s * lax.rsqrt(v + EPS)
        h1_ref[:, sl * K:(sl + 1) * K, :] = jnp.maximum(
            (slab - m) * a + bet_s, 0.0)

    h2 = lax.dot_general(h1_ref[...].reshape(B * _SB * K, OF), w2_ref[...],
                         (((1,), (1,)), ((), ())),
                         preferred_element_type=jnp.float32) + b2_ref[...]
    h1_ref[...] = h2.reshape(B, _SB * K, OF)

    for sl in range(_SB):
        slab = h1_ref[:, sl * K:(sl + 1) * K, :]
        m, v = _stats(slab)
        gam_s = lax.slice(gam, (0, 0, sl), (1, 1, sl + 1))
        bet_s = lax.slice(bet, (0, 0, sl), (1, 1, sl + 1))
        a = gam_s * lax.rsqrt(v + EPS)
        hb2 = jnp.maximum((slab - m) * a + bet_s, 0.0)
        out_ref[:, sl, :] = jnp.max(hb2, axis=1)


def _mlp(gx, xc, w1a, w1b, w2, b1, b2, gam, bet):
    nblk = S // _SB
    return pl.pallas_call(
        _mlp_body,
        grid=(nblk,),
        in_specs=[
            pl.BlockSpec((B, _SB * K, FI), lambda i: (0, i, 0)),
            pl.BlockSpec((1, _SB * K, FI), lambda i: (i % (B * S // (_SB * K)), 0, 0)),
            pl.BlockSpec((OF, FI), lambda i: (0, 0)),
            pl.BlockSpec((OF, FI), lambda i: (0, 0)),
            pl.BlockSpec((OF, OF), lambda i: (0, 0)),
            pl.BlockSpec((1, OF), lambda i: (0, 0)),
            pl.BlockSpec((1, OF), lambda i: (0, 0)),
            pl.BlockSpec((1, 1, _SB), lambda i: (i, 0, 0)),
            pl.BlockSpec((1, 1, _SB), lambda i: (i, 0, 0)),
        ],
        out_specs=pl.BlockSpec((B, _SB, OF), lambda i: (0, i, 0)),
        out_shape=jax.ShapeDtypeStruct((B, S, OF), jnp.float32),
        scratch_shapes=[pltpu.VMEM((B, _SB * K, OF), jnp.float32)],
    )(gx, xc, w1a, w1b, w2, b1, b2, gam, bet)


def kernel(x, W1, b1, W2, b2, gamma, beta):
    xf = x.reshape(B * S, FI)
    gidx = _knn(x).transpose(0, 2, 1)
    gx = _make_gather()(xf, gidx.reshape(-1))

    w1a, w1b = W1[:, :FI], W1[:, FI:]
    out = _mlp(
        gx.reshape(B, S * K, FI),
        xf.reshape(B * S // (_SB * K), _SB * K, FI),
        w1a, w1b, W2,
        b1.reshape(1, OF), b2.reshape(1, OF),
        gamma.reshape(S // _SB, 1, _SB), beta.reshape(S // _SB, 1, _SB),
    )
    return out
